# Initial kernel scaffold; baseline (speedup 1.0000x reference)
import jax
import jax.numpy as jnp
from jax.experimental import pallas as pl


def kernel(logits, labels, label_idxs, epoch, adaptive_labels):
    raise NotImplementedError("write your pallas kernel here")


# speedup vs baseline: 1.3195x; 1.3195x over previous
import jax
import jax.numpy as jnp
from jax import lax
from jax.experimental import pallas as pl
from jax.experimental.pallas import tpu as pltpu

N_TRAIN = 1000000
NUM_CLASSES = 100
BATCH = 4096
MOMENTUM = 0.9
START_EPOCH = 60


def _gather_body(idx_ref, table_ref, out_ref, sem):
  def body(i, _):
    s = idx_ref[i]
    pltpu.make_async_copy(
        table_ref.at[pl.ds(s, 1)], out_ref.at[pl.ds(i, 1)], sem).start()
    return 0

  lax.fori_loop(0, BATCH, body, 0, unroll=8)
  pltpu.make_async_copy(table_ref.at[pl.ds(0, BATCH)], out_ref, sem).wait()


_tc_gather = pl.pallas_call(
    _gather_body,
    grid_spec=pltpu.PrefetchScalarGridSpec(
        num_scalar_prefetch=1,
        in_specs=[pl.BlockSpec(memory_space=pl.ANY)],
        out_specs=pl.BlockSpec(memory_space=pl.ANY),
        scratch_shapes=[pltpu.SemaphoreType.DMA],
    ),
    out_shape=jax.ShapeDtypeStruct((BATCH, NUM_CLASSES), jnp.float32),
)


def _tc_dense_body(logits_ref, old_ref, new_ref, loss_ref):
  x = logits_ref[...]
  m = jnp.max(x, axis=1, keepdims=True)
  z = x - m
  e = jnp.exp(z)
  s = jnp.sum(e, axis=1, keepdims=True)
  probs = e / s
  logp = z - jnp.log(s)
  new = MOMENTUM * old_ref[...] + (1.0 - MOMENTUM) * probs
  new_ref[...] = new
  loss_sum = -jnp.sum(new * logp)
  w_sum = jnp.sum(jnp.max(new, axis=1))
  loss_ref[...] = (loss_sum / w_sum).reshape(1, 1)


_tc_dense = pl.pallas_call(
    _tc_dense_body,
    out_shape=(
        jax.ShapeDtypeStruct((BATCH, NUM_CLASSES), jnp.float32),
        jax.ShapeDtypeStruct((1, 1), jnp.float32),
    ),
)


def _scatter_body(idx_ref, table_ref, rows_ref, out_ref, sem):
  del table_ref
  def body(i, _):
    s = idx_ref[i]
    pltpu.make_async_copy(
        rows_ref.at[pl.ds(i, 1)], out_ref.at[pl.ds(s, 1)], sem).start()
    return 0

  lax.fori_loop(0, BATCH, body, 0, unroll=8)
  pltpu.make_async_copy(rows_ref, out_ref.at[pl.ds(0, BATCH)], sem).wait()


_tc_scatter = pl.pallas_call(
    _scatter_body,
    grid_spec=pltpu.PrefetchScalarGridSpec(
        num_scalar_prefetch=1,
        in_specs=[
            pl.BlockSpec(memory_space=pl.ANY),
            pl.BlockSpec(memory_space=pl.ANY),
        ],
        out_specs=pl.BlockSpec(memory_space=pl.ANY),
        scratch_shapes=[pltpu.SemaphoreType.DMA],
    ),
    out_shape=jax.ShapeDtypeStruct((N_TRAIN, NUM_CLASSES), jnp.float32),
    input_output_aliases={1: 0},
)


def kernel(logits, labels, label_idxs, epoch, adaptive_labels):
  def ce_branch(_):
    log_p = jax.nn.log_softmax(logits, axis=1)
    ce = -jnp.take_along_axis(log_p, labels[:, None].astype(jnp.int32),
                              axis=1)[:, 0]
    return jnp.mean(ce), adaptive_labels

  def sat_branch(_):
    old_rows = _tc_gather(label_idxs, adaptive_labels)
    new_rows, loss = _tc_dense(logits, old_rows)
    new_table = _tc_scatter(label_idxs, adaptive_labels, new_rows)
    return loss[0, 0], new_table

  return jax.lax.cond(epoch < START_EPOCH, ce_branch, sat_branch, None)


# speedup vs baseline: 3.0251x; 2.2926x over previous
import functools

import jax
import jax.numpy as jnp
from jax import lax
from jax.experimental import pallas as pl
from jax.experimental.pallas import tpu as pltpu
from jax.experimental.pallas import tpu_sc as plsc

N_TRAIN = 1000000
NUM_CLASSES = 100
BATCH = 4096
MOMENTUM = 0.9
START_EPOCH = 60

C = NUM_CLASSES
CP = 128
W = 512
NWIN = 1953
BASE64 = NWIN * W
NC_SC, NS_SC = 2, 16
NTILES = NC_SC * NS_SC
PBCAP = 128


def _tc_pre_body(logits_ref, probs_ref, logp_ref):
  x = logits_ref[...]
  colv = lax.broadcasted_iota(jnp.int32, (BATCH, CP), 1)
  xp = jnp.pad(x, ((0, 0), (0, CP - C)), constant_values=-1e30)
  m = jnp.max(xp, axis=1, keepdims=True)
  z = xp - m
  e = jnp.exp(z)
  s = jnp.sum(e, axis=1, keepdims=True)
  probs_ref[...] = e / s
  logp_ref[...] = jnp.where(colv < C, z - jnp.log(s), 0.0)


_tc_pre = pl.pallas_call(
    _tc_pre_body,
    out_shape=(
        jax.ShapeDtypeStruct((BATCH, CP), jnp.float32),
        jax.ShapeDtypeStruct((BATCH, CP), jnp.float32),
    ),
)


def _make_sc_sweep():
  mesh_sc = plsc.VectorSubcoreMesh(
      core_axis_name="c", subcore_axis_name="s",
      num_cores=NC_SC, num_subcores=NS_SC)

  @functools.partial(
      pl.kernel,
      out_type=(
          jax.ShapeDtypeStruct((C, N_TRAIN), jnp.float32),
          jax.ShapeDtypeStruct((NTILES, 16), jnp.float32),
      ),
      mesh=mesh_sc,
      scratch_types=[
          pltpu.VMEM((BATCH + 16,), jnp.int32),
          pltpu.VMEM((BATCH + 16,), jnp.int32),
          pltpu.VMEM((BATCH + 16,), jnp.int32),
          pltpu.VMEM((C, W), jnp.float32),
          pltpu.VMEM((PBCAP, CP), jnp.float32),
          pltpu.VMEM((PBCAP, CP), jnp.float32),
          pltpu.VMEM((16,), jnp.float32),
          pltpu.SMEM((PBCAP,), jnp.int32),
          pltpu.SemaphoreType.DMA,
      ],
      compiler_params=pltpu.CompilerParams(needs_layout_passes=False),
  )
  def sc_sweep(tT_hbm, idx_hbm, probs_hbm, logp_hbm,
               outT_hbm, part_hbm,
               idx_v, my_idx, my_gid, win_v, pb_v, lb_v, st_v, sl_s, rsem):
    wid = lax.axis_index("s") * NC_SC + lax.axis_index("c")
    lanes = lax.iota(jnp.int32, 16)
    pltpu.sync_copy(idx_hbm, idx_v.at[pl.ds(0, BATCH)])

    def compact(cc, nm):
      iv = idx_v[pl.ds(cc * 16, 16)]
      mine = jnp.logical_and((iv >> 9) & (NTILES - 1) == wid, iv < BASE64)
      plsc.store_compressed(my_idx.at[pl.ds(nm, 16)], iv, mask=mine)
      plsc.store_compressed(my_gid.at[pl.ds(nm, 16)], cc * 16 + lanes, mask=mine)
      return nm + jnp.max(plsc.all_reduce_population_count(mine))

    nmine = lax.fori_loop(0, BATCH // 16, compact, 0)
    nch = (nmine + 15) >> 4

    def window_body(wi, carry):
      loss_acc, w_acc = carry
      o = (wid + wi * NTILES) * W
      pltpu.sync_copy(tT_hbm.at[:, pl.ds(o, W)], win_v)

      def scan_chunk(cc, k):
        iv = my_idx[pl.ds(cc * 16, 16)]
        gv = my_gid[pl.ds(cc * 16, 16)]
        valid = (cc * 16 + lanes) < nmine
        inwin = jnp.logical_and(
            valid, jnp.logical_and(iv >= o, iv < o + W))
        cnt = jnp.max(plsc.all_reduce_population_count(inwin))

        def lane_loop(_, state):
          k2, m = state
          j = jnp.max(plsc.all_reduce_ffs(m))
          onehot = (lanes == j).astype(jnp.int32)
          val = jnp.sum(iv * onehot)
          gid = jnp.sum(gv * onehot)
          pltpu.make_async_copy(
              probs_hbm.at[pl.ds(gid, 1)],
              pb_v.at[pl.ds(k2, 1)], rsem).start()
          pltpu.make_async_copy(
              logp_hbm.at[pl.ds(gid, 1)],
              lb_v.at[pl.ds(k2, 1)], rsem).start()
          sl_s[k2] = val - o
          return k2 + 1, jnp.logical_and(m, lanes != j)

        k2, _ = lax.fori_loop(0, cnt, lane_loop, (k, inwin))
        return k2

      nb = lax.fori_loop(0, nch, scan_chunk, 0)

      def drain(_, c):
        pltpu.make_async_copy(
            probs_hbm.at[pl.ds(0, 1)], pb_v.at[pl.ds(0, 1)], rsem).wait()
        pltpu.make_async_copy(
            logp_hbm.at[pl.ds(0, 1)], lb_v.at[pl.ds(0, 1)], rsem).wait()
        return c

      lax.fori_loop(0, nb, drain, 0)

      def rmw(k, carry2):
        la, wa = carry2
        col = jnp.full((16,), sl_s[k], dtype=jnp.int32)
        dotv = jnp.zeros((16,), jnp.float32)
        wmax = jnp.full((16,), -jnp.inf, dtype=jnp.float32)
        for cc in range(7):
          rows = cc * 16 + lanes
          msk = rows < C
          old16 = plsc.load_gather(win_v, [rows, col], mask=msk)
          pk = pb_v[k, pl.ds(cc * 16, 16)]
          lk = lb_v[k, pl.ds(cc * 16, 16)]
          new16 = MOMENTUM * old16 + (1.0 - MOMENTUM) * pk
          plsc.store_scatter(win_v, [rows, col], new16, mask=msk)
          dotv = dotv + new16 * lk * jnp.where(msk, 1.0, 0.0)
          wmax = jnp.maximum(wmax, jnp.where(msk, new16, -jnp.inf))
        return la - jnp.sum(dotv), wa + jnp.max(wmax)

      loss_acc, w_acc = lax.fori_loop(0, nb, rmw, (loss_acc, w_acc))

      pltpu.sync_copy(win_v, outT_hbm.at[:, pl.ds(o, W)])
      return loss_acc, w_acc

    nwin_mine = (NWIN - 1 - wid) // NTILES + 1
    loss_acc, w_acc = lax.fori_loop(
        0, nwin_mine, window_body, (jnp.float32(0.0), jnp.float32(0.0)))

    st_v[...] = jnp.where(lanes == 0, loss_acc,
                          jnp.where(lanes == 1, w_acc, 0.0))
    pltpu.sync_copy(st_v, part_hbm.at[wid])

  return sc_sweep


_sc_sweep = _make_sc_sweep()


def _tc_tail_body(alias_ref, tblk_ref, probs_ref, logp_ref, idx_ref,
                  parts_ref, outblk_ref, loss_ref):
  del alias_ref
  blk = tblk_ref[...]
  idxc = idx_ref[...]
  col = lax.broadcasted_iota(jnp.int32, (1, 128), 1) + BASE64
  m_scatter = (idxc == col).astype(jnp.float32)
  old = lax.dot_general(m_scatter, blk, (((1,), (1,)), ((), ())))
  inb = (idxc >= BASE64).astype(jnp.float32)
  new = MOMENTUM * old + (1.0 - MOMENTUM) * probs_ref[:, :C]
  scat = lax.dot_general(new, m_scatter, (((0,), (0,)), ((), ())))
  hit = jnp.max(m_scatter, axis=0, keepdims=True)
  outblk_ref[...] = jnp.where(hit > 0, scat, blk)
  lp = -jnp.sum(new * logp_ref[:, :C], axis=1, keepdims=True)
  wv = jnp.max(new, axis=1, keepdims=True)
  loss_sum = jnp.sum(parts_ref[:, 0:1]) + jnp.sum(lp * inb)
  w_sum = jnp.sum(parts_ref[:, 1:2]) + jnp.sum(wv * inb)
  loss_ref[...] = (loss_sum / w_sum).reshape(1, 1)


_tc_tail = pl.pallas_call(
    _tc_tail_body,
    grid=(1,),
    in_specs=[
        pl.BlockSpec(memory_space=pl.ANY),
        pl.BlockSpec((C, 128), lambda i: (0, BASE64 // 128)),
        pl.BlockSpec((BATCH, CP), lambda i: (0, 0)),
        pl.BlockSpec((BATCH, CP), lambda i: (0, 0)),
        pl.BlockSpec((BATCH, 1), lambda i: (0, 0)),
        pl.BlockSpec((NTILES, 16), lambda i: (0, 0)),
    ],
    out_specs=(
        pl.BlockSpec((C, 128), lambda i: (0, BASE64 // 128)),
        pl.BlockSpec((1, 1), lambda i: (0, 0)),
    ),
    out_shape=(
        jax.ShapeDtypeStruct((C, N_TRAIN), jnp.float32),
        jax.ShapeDtypeStruct((1, 1), jnp.float32),
    ),
    input_output_aliases={0: 0},
)


def kernel(logits, labels, label_idxs, epoch, adaptive_labels):
  def ce_branch(_):
    log_p = jax.nn.log_softmax(logits, axis=1)
    ce = -jnp.take_along_axis(log_p, labels[:, None].astype(jnp.int32),
                              axis=1)[:, 0]
    return jnp.mean(ce), adaptive_labels

  def sat_branch(_):
    probs_pad, logp_pad = _tc_pre(logits)
    tT = jnp.transpose(adaptive_labels)
    outT, parts = _sc_sweep(tT, label_idxs, probs_pad, logp_pad)
    outT2, loss = _tc_tail(outT, tT, probs_pad, logp_pad,
                           label_idxs.reshape(BATCH, 1), parts)
    return loss[0, 0], jnp.transpose(outT2)

  return jax.lax.cond(epoch < START_EPOCH, ce_branch, sat_branch, None)


# speedup vs baseline: 3.9334x; 1.3002x over previous
import functools

import jax
import jax.numpy as jnp
from jax import lax
from jax.experimental import pallas as pl
from jax.experimental.pallas import tpu as pltpu
from jax.experimental.pallas import tpu_sc as plsc

N_TRAIN = 1000000
NUM_CLASSES = 100
BATCH = 4096
MOMENTUM = 0.9
START_EPOCH = 60

C = NUM_CLASSES
CP = 128
W = 512
NWIN = 1953
BASE64 = NWIN * W
NC_SC, NS_SC = 2, 16
NTILES = NC_SC * NS_SC
PBCAP = 32
WBYTES = C * W * 4


def _tc_pre_body(logits_ref, probs_ref, logp_ref):
  x = logits_ref[...]
  colv = lax.broadcasted_iota(jnp.int32, (BATCH, CP), 1)
  xp = jnp.pad(x, ((0, 0), (0, CP - C)), constant_values=-1e30)
  m = jnp.max(xp, axis=1, keepdims=True)
  z = xp - m
  e = jnp.exp(z)
  s = jnp.sum(e, axis=1, keepdims=True)
  probs_ref[...] = e / s
  logp_ref[...] = jnp.where(colv < C, z - jnp.log(s), 0.0)


_tc_pre = pl.pallas_call(
    _tc_pre_body,
    out_shape=(
        jax.ShapeDtypeStruct((BATCH, CP), jnp.float32),
        jax.ShapeDtypeStruct((BATCH, CP), jnp.float32),
    ),
)


def _make_sc_sweep():
  mesh_sc = plsc.VectorSubcoreMesh(
      core_axis_name="c", subcore_axis_name="s",
      num_cores=NC_SC, num_subcores=NS_SC)

  @functools.partial(
      pl.kernel,
      out_type=(
          jax.ShapeDtypeStruct((C, N_TRAIN), jnp.float32),
          jax.ShapeDtypeStruct((NTILES, 16), jnp.float32),
      ),
      mesh=mesh_sc,
      scratch_types=[
          pltpu.VMEM((BATCH + 16,), jnp.int32),
          pltpu.VMEM((BATCH + 16,), jnp.int32),
          pltpu.VMEM((BATCH + 16,), jnp.int32),
          pltpu.VMEM((C, W), jnp.float32),
          pltpu.VMEM((C, W), jnp.float32),
          pltpu.VMEM((PBCAP, CP), jnp.float32),
          pltpu.VMEM((PBCAP, CP), jnp.float32),
          pltpu.VMEM((16,), jnp.float32),
          pltpu.VMEM((16,), jnp.float32),
          pltpu.SMEM((PBCAP,), jnp.int32),
          pltpu.SemaphoreType.DMA,
          pltpu.SemaphoreType.DMA,
          pltpu.SemaphoreType.DMA,
      ],
      compiler_params=pltpu.CompilerParams(needs_layout_passes=False),
  )
  def sc_sweep(tT_hbm, idx_hbm, probs_hbm, logp_hbm,
               outT_hbm, part_hbm,
               idx_v, my_idx, my_gid, win0_v, win1_v, pb_v, lb_v, st_v,
               acc_v, sl_s, isem, osem, rsem):
    wid = lax.axis_index("s") * NC_SC + lax.axis_index("c")
    lanes = lax.iota(jnp.int32, 16)
    acc_v[...] = jnp.zeros((16,), jnp.float32)
    pltpu.sync_copy(idx_hbm, idx_v.at[pl.ds(0, BATCH)])

    def compact(cc, nm):
      iv = idx_v[pl.ds(cc * 16, 16)]
      mine = jnp.logical_and((iv >> 9) & (NTILES - 1) == wid, iv < BASE64)
      plsc.store_compressed(my_idx.at[pl.ds(nm, 16)], iv, mask=mine)
      plsc.store_compressed(my_gid.at[pl.ds(nm, 16)], cc * 16 + lanes, mask=mine)
      return nm + jnp.max(plsc.all_reduce_population_count(mine))

    nmine = lax.fori_loop(0, BATCH // 16, compact, 0)
    nch = (nmine + 15) >> 4

    nwin_mine = (NWIN - 1 - wid) // NTILES + 1

    def start_in(o_next, buf):
      pltpu.make_async_copy(tT_hbm.at[:, pl.ds(o_next, W)], buf, isem).start()

    def wait_in():
      pltpu.make_async_copy(tT_hbm.at[:, pl.ds(0, W)], win0_v, isem).wait()

    def wait_out():
      pltpu.make_async_copy(tT_hbm.at[:, pl.ds(0, W)], win0_v, osem).wait()

    start_in(wid * W, win0_v)

    def window_body(wi, carry):
      o = (wid + wi * NTILES) * W

      def scan_chunk(cc, k):
        iv = my_idx[pl.ds(cc * 16, 16)]
        gv = my_gid[pl.ds(cc * 16, 16)]
        valid = (cc * 16 + lanes) < nmine
        inwin = jnp.logical_and(
            valid, jnp.logical_and(iv >= o, iv < o + W))
        cnt = jnp.max(plsc.all_reduce_population_count(inwin))

        def lane_loop(_, state):
          k2, m = state
          j = jnp.max(plsc.all_reduce_ffs(m))
          onehot = (lanes == j).astype(jnp.int32)
          val = jnp.sum(iv * onehot)
          gid = jnp.sum(gv * onehot)
          pltpu.make_async_copy(
              probs_hbm.at[pl.ds(gid, 1)],
              pb_v.at[pl.ds(k2, 1)], rsem).start()
          pltpu.make_async_copy(
              logp_hbm.at[pl.ds(gid, 1)],
              lb_v.at[pl.ds(k2, 1)], rsem).start()
          sl_s[k2] = val - o
          return k2 + 1, jnp.logical_and(m, lanes != j)

        k2, _ = lax.fori_loop(0, cnt, lane_loop, (k, inwin))
        return k2

      nb = lax.fori_loop(0, nch, scan_chunk, 0)

      @pl.when(wi + 1 < nwin_mine)
      def _():
        @pl.when(wi >= 1)
        def _():
          wait_out()

        o_next = o + NTILES * W

        @pl.when(wi % 2 == 0)
        def _():
          start_in(o_next, win1_v)

        @pl.when(wi % 2 == 1)
        def _():
          start_in(o_next, win0_v)

      def drain(_, c):
        pltpu.make_async_copy(
            probs_hbm.at[pl.ds(0, 1)], pb_v.at[pl.ds(0, 1)], rsem).wait()
        pltpu.make_async_copy(
            logp_hbm.at[pl.ds(0, 1)], lb_v.at[pl.ds(0, 1)], rsem).wait()
        return c

      lax.fori_loop(0, nb, drain, 0)
      wait_in()

      def run_phase_b(buf):
        def rmw(k, c2):
          col = jnp.full((16,), sl_s[k], dtype=jnp.int32)
          dotv = jnp.zeros((16,), jnp.float32)
          wmax = jnp.full((16,), -jnp.inf, dtype=jnp.float32)
          for cc in range(7):
            rows = cc * 16 + lanes
            msk = rows < C
            old16 = plsc.load_gather(buf, [rows, col], mask=msk)
            pk = pb_v[k, pl.ds(cc * 16, 16)]
            lk = lb_v[k, pl.ds(cc * 16, 16)]
            new16 = MOMENTUM * old16 + (1.0 - MOMENTUM) * pk
            plsc.store_scatter(buf, [rows, col], new16, mask=msk)
            dotv = dotv + new16 * lk * jnp.where(msk, 1.0, 0.0)
            wmax = jnp.maximum(wmax, jnp.where(msk, new16, -jnp.inf))
          acc_v[...] = (acc_v[...]
                        + jnp.where(lanes == 0, -jnp.sum(dotv), 0.0)
                        + jnp.where(lanes == 1, jnp.max(wmax), 0.0))
          return c2

        lax.fori_loop(0, nb, rmw, 0)
        pltpu.make_async_copy(buf, outT_hbm.at[:, pl.ds(o, W)], osem).start()

      @pl.when(wi % 2 == 0)
      def _():
        run_phase_b(win0_v)

      @pl.when(wi % 2 == 1)
      def _():
        run_phase_b(win1_v)

      return carry

    lax.fori_loop(0, nwin_mine, window_body, 0)
    wait_out()
    wait_out()

    st_v[...] = acc_v[...]
    pltpu.sync_copy(st_v, part_hbm.at[wid])

  return sc_sweep


_sc_sweep = _make_sc_sweep()


def _tc_tail_body(alias_ref, tblk_ref, probs_ref, logp_ref, idx_ref,
                  parts_ref, outblk_ref, loss_ref):
  del alias_ref
  blk = tblk_ref[...]
  idxc = idx_ref[...]
  col = lax.broadcasted_iota(jnp.int32, (1, 128), 1) + BASE64
  m_scatter = (idxc == col).astype(jnp.float32)
  old = lax.dot_general(m_scatter, blk, (((1,), (1,)), ((), ())))
  inb = (idxc >= BASE64).astype(jnp.float32)
  new = MOMENTUM * old + (1.0 - MOMENTUM) * probs_ref[:, :C]
  scat = lax.dot_general(new, m_scatter, (((0,), (0,)), ((), ())))
  hit = jnp.max(m_scatter, axis=0, keepdims=True)
  outblk_ref[...] = jnp.where(hit > 0, scat, blk)
  lp = -jnp.sum(new * logp_ref[:, :C], axis=1, keepdims=True)
  wv = jnp.max(new, axis=1, keepdims=True)
  loss_sum = jnp.sum(parts_ref[:, 0:1]) + jnp.sum(lp * inb)
  w_sum = jnp.sum(parts_ref[:, 1:2]) + jnp.sum(wv * inb)
  loss_ref[...] = (loss_sum / w_sum).reshape(1, 1)


_tc_tail = pl.pallas_call(
    _tc_tail_body,
    grid=(1,),
    in_specs=[
        pl.BlockSpec(memory_space=pl.ANY),
        pl.BlockSpec((C, 128), lambda i: (0, BASE64 // 128)),
        pl.BlockSpec((BATCH, CP), lambda i: (0, 0)),
        pl.BlockSpec((BATCH, CP), lambda i: (0, 0)),
        pl.BlockSpec((BATCH, 1), lambda i: (0, 0)),
        pl.BlockSpec((NTILES, 16), lambda i: (0, 0)),
    ],
    out_specs=(
        pl.BlockSpec((C, 128), lambda i: (0, BASE64 // 128)),
        pl.BlockSpec((1, 1), lambda i: (0, 0)),
    ),
    out_shape=(
        jax.ShapeDtypeStruct((C, N_TRAIN), jnp.float32),
        jax.ShapeDtypeStruct((1, 1), jnp.float32),
    ),
    input_output_aliases={0: 0},
)


def kernel(logits, labels, label_idxs, epoch, adaptive_labels):
  def ce_branch(_):
    log_p = jax.nn.log_softmax(logits, axis=1)
    ce = -jnp.take_along_axis(log_p, labels[:, None].astype(jnp.int32),
                              axis=1)[:, 0]
    return jnp.mean(ce), adaptive_labels

  def sat_branch(_):
    probs_pad, logp_pad = _tc_pre(logits)
    tT = jnp.transpose(adaptive_labels)
    outT, parts = _sc_sweep(tT, label_idxs, probs_pad, logp_pad)
    outT2, loss = _tc_tail(outT, tT, probs_pad, logp_pad,
                           label_idxs.reshape(BATCH, 1), parts)
    return loss[0, 0], jnp.transpose(outT2)

  return jax.lax.cond(epoch < START_EPOCH, ce_branch, sat_branch, None)
